# baseline (speedup 1.0000x reference)
import functools

import jax
import jax.numpy as jnp
from jax.experimental import pallas as pl

_D = 128
_R = 64
_HOPS = 2
_EBLK = 1000


def _proj_kernel(x_ref, r_ref, w_ref, ab_ref):
    q1 = jnp.dot(r_ref[:], w_ref[:, :_D], preferred_element_type=jnp.float32)
    q2 = jnp.dot(r_ref[:], w_ref[:, _D:], preferred_element_type=jnp.float32)
    q = jnp.concatenate([q1, q2], axis=0)
    ab_ref[:] = jnp.dot(x_ref[:], q.T, preferred_element_type=jnp.float32)


def _score_kernel(a_ref, b_ref, c_ref, e_ref):
    s = a_ref[:] + b_ref[:] + c_ref[:]
    e_ref[:] = jnp.where(s >= 0, s, 0.2 * s)


def _exp_kernel(e_ref, m_ref, out_ref):
    out_ref[:] = jnp.exp(e_ref[:] - m_ref[:])


def _msg_kernel(xt_ref, ee_ref, sh_ref, out_ref):
    attn = ee_ref[:] / (sh_ref[:] + 1e-16)
    out_ref[:] = xt_ref[:] * attn


def _norm_kernel(agg_ref, x_ref, out_ref):
    v = agg_ref[:] + x_ref[:]
    n = jnp.sqrt(jnp.sum(v * v, axis=-1, keepdims=True))
    out_ref[:] = v / jnp.maximum(n, 1e-12)


def _hop(x, relation_emb, head, tail, edge_type, fc_w, fc_b):
    n = x.shape[0]
    e_cnt = head.shape[0]

    ab = pl.pallas_call(
        _proj_kernel,
        out_shape=jax.ShapeDtypeStruct((n, 2 * _R), jnp.float32),
    )(x, relation_emb, fc_w)

    c = relation_emb @ fc_b

    ae = ab[head, edge_type]
    be = ab[tail, edge_type + _R]
    ce = c[edge_type]

    rows = (e_cnt + _D - 1) // _D
    epad = rows * _D

    def _to2d(v):
        return jnp.pad(v, (0, epad - e_cnt)).reshape(rows, _D)

    e2d = pl.pallas_call(
        _score_kernel,
        out_shape=jax.ShapeDtypeStruct((rows, _D), jnp.float32),
    )(_to2d(ae), _to2d(be), _to2d(ce))
    e = e2d.reshape(epad)[:e_cnt]

    m = jax.ops.segment_max(e, head, num_segments=n)
    m = jnp.where(jnp.isfinite(m), m, 0.0)

    ee2d = pl.pallas_call(
        _exp_kernel,
        out_shape=jax.ShapeDtypeStruct((rows, _D), jnp.float32),
    )(e2d, _to2d(m[head]))
    ee = ee2d.reshape(epad)[:e_cnt]

    s = jax.ops.segment_sum(ee, head, num_segments=n)

    eb = (e_cnt + _EBLK - 1) // _EBLK
    epad2 = eb * _EBLK
    xt = jnp.pad(x[tail], ((0, epad2 - e_cnt), (0, 0)))
    ee_c = jnp.pad(ee, (0, epad2 - e_cnt)).reshape(epad2, 1)
    sh_c = jnp.pad(s[head], (0, epad2 - e_cnt)).reshape(epad2, 1)

    msg = pl.pallas_call(
        _msg_kernel,
        grid=(eb,),
        in_specs=[
            pl.BlockSpec((_EBLK, _D), lambda i: (i, 0)),
            pl.BlockSpec((_EBLK, 1), lambda i: (i, 0)),
            pl.BlockSpec((_EBLK, 1), lambda i: (i, 0)),
        ],
        out_specs=pl.BlockSpec((_EBLK, _D), lambda i: (i, 0)),
        out_shape=jax.ShapeDtypeStruct((epad2, _D), jnp.float32),
    )(xt, ee_c, sh_c)[:e_cnt]

    agg = jax.ops.segment_sum(msg, head, num_segments=n)

    return pl.pallas_call(
        _norm_kernel,
        out_shape=jax.ShapeDtypeStruct((n, _D), jnp.float32),
    )(agg, x)


@jax.jit
def kernel(entity_emb, relation_emb, edge_index, edge_type, fc_w, fc_b):
    head = edge_index[0]
    tail = edge_index[1]
    x = entity_emb
    for _ in range(_HOPS):
        x = _hop(x, relation_emb, head, tail, edge_type, fc_w, fc_b)
    return x
